# Initial kernel scaffold; baseline (speedup 1.0000x reference)
import jax
import jax.numpy as jnp
from jax.experimental import pallas as pl


def kernel(feature, adj, W1, b1, W2, b2):
    raise NotImplementedError("write your pallas kernel here")


# speedup vs baseline: 1.3449x; 1.3449x over previous
import functools

import jax
import jax.numpy as jnp
from jax.experimental import pallas as pl

N = 10000
D_IN = 128
D_HID = 128
N_CLASS = 16
ROW_BLK = 200
GRID = N // ROW_BLK


def _pass1_kernel(adj_ref, x_ref, xself_ref, w1_ref, b1_ref, w2_ref, b2_ref,
                  yself_ref, yneigh_ref):
    adj = adj_ref[...]
    x = x_ref[...]
    deg = jnp.sum(adj, axis=1, keepdims=True)
    deg = jnp.maximum(deg, 1e-12)
    acc = jax.lax.dot_general(
        adj, x, (((1,), (0,)), ((), ())), preferred_element_type=jnp.float32)
    neigh = acc / deg
    xs = xself_ref[...]
    w1 = w1_ref[...]
    h = (jax.lax.dot_general(xs, w1[:D_IN], (((1,), (0,)), ((), ())),
                             preferred_element_type=jnp.float32)
         + jax.lax.dot_general(neigh, w1[D_IN:], (((1,), (0,)), ((), ())),
                               preferred_element_type=jnp.float32)
         + b1_ref[...])
    h = jnp.maximum(h, 0.0)
    w2 = w2_ref[...]
    yself_ref[...] = jax.lax.dot_general(
        h, w2[:D_HID], (((1,), (0,)), ((), ())),
        preferred_element_type=jnp.float32) + b2_ref[...]
    yneigh_ref[...] = jax.lax.dot_general(
        h, w2[D_HID:], (((1,), (0,)), ((), ())),
        preferred_element_type=jnp.float32)


def _pass2_kernel(adj_ref, yneigh_ref, yself_ref, out_ref):
    adj = adj_ref[...]
    deg = jnp.sum(adj, axis=1, keepdims=True)
    deg = jnp.maximum(deg, 1e-12)
    acc = jax.lax.dot_general(
        adj, yneigh_ref[...], (((1,), (0,)), ((), ())),
        preferred_element_type=jnp.float32)
    logits = yself_ref[...] + acc / deg
    m = jnp.max(logits, axis=1, keepdims=True)
    s = logits - m
    lse = jnp.log(jnp.sum(jnp.exp(s), axis=1, keepdims=True))
    out_ref[...] = s - lse


@functools.partial(jax.jit, static_argnames=("interpret",))
def kernel(feature, adj, W1, b1, W2, b2, interpret=False):
    b1r = b1.reshape(1, D_HID)
    b2r = b2.reshape(1, N_CLASS)

    yself, yneigh = pl.pallas_call(
        _pass1_kernel,
        grid=(GRID,),
        in_specs=[
            pl.BlockSpec((ROW_BLK, N), lambda i: (i, 0)),
            pl.BlockSpec((N, D_IN), lambda i: (0, 0)),
            pl.BlockSpec((ROW_BLK, D_IN), lambda i: (i, 0)),
            pl.BlockSpec((2 * D_IN, D_HID), lambda i: (0, 0)),
            pl.BlockSpec((1, D_HID), lambda i: (0, 0)),
            pl.BlockSpec((2 * D_HID, N_CLASS), lambda i: (0, 0)),
            pl.BlockSpec((1, N_CLASS), lambda i: (0, 0)),
        ],
        out_specs=[
            pl.BlockSpec((ROW_BLK, N_CLASS), lambda i: (i, 0)),
            pl.BlockSpec((ROW_BLK, N_CLASS), lambda i: (i, 0)),
        ],
        out_shape=[
            jax.ShapeDtypeStruct((N, N_CLASS), jnp.float32),
            jax.ShapeDtypeStruct((N, N_CLASS), jnp.float32),
        ],
        interpret=interpret,
    )(adj, feature, feature, W1, b1r, W2, b2r)

    out = pl.pallas_call(
        _pass2_kernel,
        grid=(GRID,),
        in_specs=[
            pl.BlockSpec((ROW_BLK, N), lambda i: (i, 0)),
            pl.BlockSpec((N, N_CLASS), lambda i: (0, 0)),
            pl.BlockSpec((ROW_BLK, N_CLASS), lambda i: (i, 0)),
        ],
        out_specs=pl.BlockSpec((ROW_BLK, N_CLASS), lambda i: (i, 0)),
        out_shape=jax.ShapeDtypeStruct((N, N_CLASS), jnp.float32),
        interpret=interpret,
    )(adj, yneigh, yself)
    return out


# speedup vs baseline: 1.3731x; 1.0209x over previous
import functools

import jax
import jax.numpy as jnp
from jax.experimental import pallas as pl

N = 10000
D_IN = 128
D_HID = 128
N_CLASS = 16
ROW_BLK = 200
GRID = N // ROW_BLK


def _pass1_kernel(adj_ref, x_ref, xself_ref, w1_ref, b1_ref, w2_ref, b2_ref,
                  yself_ref, yneigh_ref):
    adj = adj_ref[...]
    x = x_ref[...]
    deg = jnp.sum(adj, axis=1, keepdims=True)
    deg = jnp.maximum(deg, 1e-12)
    acc = jax.lax.dot_general(
        adj.astype(jnp.bfloat16), x.astype(jnp.bfloat16),
        (((1,), (0,)), ((), ())), preferred_element_type=jnp.float32)
    neigh = acc / deg
    xs = xself_ref[...]
    w1 = w1_ref[...]
    h = (jax.lax.dot_general(xs, w1[:D_IN], (((1,), (0,)), ((), ())),
                             preferred_element_type=jnp.float32)
         + jax.lax.dot_general(neigh, w1[D_IN:], (((1,), (0,)), ((), ())),
                               preferred_element_type=jnp.float32)
         + b1_ref[...])
    h = jnp.maximum(h, 0.0)
    w2 = w2_ref[...]
    yself_ref[...] = jax.lax.dot_general(
        h, w2[:D_HID], (((1,), (0,)), ((), ())),
        preferred_element_type=jnp.float32) + b2_ref[...]
    yneigh_ref[...] = jax.lax.dot_general(
        h, w2[D_HID:], (((1,), (0,)), ((), ())),
        preferred_element_type=jnp.float32)


def _pass2_kernel(adj_ref, yneigh_ref, yself_ref, out_ref):
    adj = adj_ref[...]
    deg = jnp.sum(adj, axis=1, keepdims=True)
    deg = jnp.maximum(deg, 1e-12)
    acc = jax.lax.dot_general(
        adj, yneigh_ref[...], (((1,), (0,)), ((), ())),
        preferred_element_type=jnp.float32)
    logits = yself_ref[...] + acc / deg
    m = jnp.max(logits, axis=1, keepdims=True)
    s = logits - m
    lse = jnp.log(jnp.sum(jnp.exp(s), axis=1, keepdims=True))
    out_ref[...] = s - lse


@functools.partial(jax.jit, static_argnames=("interpret",))
def kernel(feature, adj, W1, b1, W2, b2, interpret=False):
    b1r = b1.reshape(1, D_HID)
    b2r = b2.reshape(1, N_CLASS)

    yself, yneigh = pl.pallas_call(
        _pass1_kernel,
        grid=(GRID,),
        in_specs=[
            pl.BlockSpec((ROW_BLK, N), lambda i: (i, 0)),
            pl.BlockSpec((N, D_IN), lambda i: (0, 0)),
            pl.BlockSpec((ROW_BLK, D_IN), lambda i: (i, 0)),
            pl.BlockSpec((2 * D_IN, D_HID), lambda i: (0, 0)),
            pl.BlockSpec((1, D_HID), lambda i: (0, 0)),
            pl.BlockSpec((2 * D_HID, N_CLASS), lambda i: (0, 0)),
            pl.BlockSpec((1, N_CLASS), lambda i: (0, 0)),
        ],
        out_specs=[
            pl.BlockSpec((ROW_BLK, N_CLASS), lambda i: (i, 0)),
            pl.BlockSpec((ROW_BLK, N_CLASS), lambda i: (i, 0)),
        ],
        out_shape=[
            jax.ShapeDtypeStruct((N, N_CLASS), jnp.float32),
            jax.ShapeDtypeStruct((N, N_CLASS), jnp.float32),
        ],
        interpret=interpret,
    )(adj, feature, feature, W1, b1r, W2, b2r)

    out = pl.pallas_call(
        _pass2_kernel,
        grid=(GRID,),
        in_specs=[
            pl.BlockSpec((ROW_BLK, N), lambda i: (i, 0)),
            pl.BlockSpec((N, N_CLASS), lambda i: (0, 0)),
            pl.BlockSpec((ROW_BLK, N_CLASS), lambda i: (i, 0)),
        ],
        out_specs=pl.BlockSpec((ROW_BLK, N_CLASS), lambda i: (i, 0)),
        out_shape=jax.ShapeDtypeStruct((N, N_CLASS), jnp.float32),
        interpret=interpret,
    )(adj, yneigh, yself)
    return out


# speedup vs baseline: 1.4012x; 1.0205x over previous
import functools

import jax
import jax.numpy as jnp
from jax.experimental import pallas as pl

N = 10000
D_IN = 128
D_HID = 128
N_CLASS = 16
ROW_BLK = 400
GRID = N // ROW_BLK


def _pass1_kernel(adj_ref, x_ref, xself_ref, w1_ref, b1_ref, w2_ref, b2_ref,
                  yself_ref, yneigh_ref):
    adj = adj_ref[...]
    x = x_ref[...]
    deg = jnp.sum(adj, axis=1, keepdims=True)
    deg = jnp.maximum(deg, 1e-12)
    acc = jax.lax.dot_general(
        adj.astype(jnp.bfloat16), x.astype(jnp.bfloat16),
        (((1,), (0,)), ((), ())), preferred_element_type=jnp.float32)
    neigh = acc / deg
    xs = xself_ref[...]
    w1 = w1_ref[...]
    h = (jax.lax.dot_general(xs, w1[:D_IN], (((1,), (0,)), ((), ())),
                             preferred_element_type=jnp.float32)
         + jax.lax.dot_general(neigh, w1[D_IN:], (((1,), (0,)), ((), ())),
                               preferred_element_type=jnp.float32)
         + b1_ref[...])
    h = jnp.maximum(h, 0.0)
    w2 = w2_ref[...]
    yself_ref[...] = jax.lax.dot_general(
        h, w2[:D_HID], (((1,), (0,)), ((), ())),
        preferred_element_type=jnp.float32) + b2_ref[...]
    yneigh_ref[...] = jax.lax.dot_general(
        h, w2[D_HID:], (((1,), (0,)), ((), ())),
        preferred_element_type=jnp.float32)


def _pass2_kernel(adj_ref, yneigh_ref, yself_ref, out_ref):
    adj = adj_ref[...]
    deg = jnp.sum(adj, axis=1, keepdims=True)
    deg = jnp.maximum(deg, 1e-12)
    acc = jax.lax.dot_general(
        adj, yneigh_ref[...], (((1,), (0,)), ((), ())),
        preferred_element_type=jnp.float32)
    logits = yself_ref[...] + acc / deg
    m = jnp.max(logits, axis=1, keepdims=True)
    s = logits - m
    lse = jnp.log(jnp.sum(jnp.exp(s), axis=1, keepdims=True))
    out_ref[...] = s - lse


@functools.partial(jax.jit, static_argnames=("interpret",))
def kernel(feature, adj, W1, b1, W2, b2, interpret=False):
    b1r = b1.reshape(1, D_HID)
    b2r = b2.reshape(1, N_CLASS)

    yself, yneigh = pl.pallas_call(
        _pass1_kernel,
        grid=(GRID,),
        in_specs=[
            pl.BlockSpec((ROW_BLK, N), lambda i: (i, 0)),
            pl.BlockSpec((N, D_IN), lambda i: (0, 0)),
            pl.BlockSpec((ROW_BLK, D_IN), lambda i: (i, 0)),
            pl.BlockSpec((2 * D_IN, D_HID), lambda i: (0, 0)),
            pl.BlockSpec((1, D_HID), lambda i: (0, 0)),
            pl.BlockSpec((2 * D_HID, N_CLASS), lambda i: (0, 0)),
            pl.BlockSpec((1, N_CLASS), lambda i: (0, 0)),
        ],
        out_specs=[
            pl.BlockSpec((ROW_BLK, N_CLASS), lambda i: (i, 0)),
            pl.BlockSpec((ROW_BLK, N_CLASS), lambda i: (i, 0)),
        ],
        out_shape=[
            jax.ShapeDtypeStruct((N, N_CLASS), jnp.float32),
            jax.ShapeDtypeStruct((N, N_CLASS), jnp.float32),
        ],
        interpret=interpret,
    )(adj, feature, feature, W1, b1r, W2, b2r)

    out = pl.pallas_call(
        _pass2_kernel,
        grid=(GRID,),
        in_specs=[
            pl.BlockSpec((ROW_BLK, N), lambda i: (i, 0)),
            pl.BlockSpec((N, N_CLASS), lambda i: (0, 0)),
            pl.BlockSpec((ROW_BLK, N_CLASS), lambda i: (i, 0)),
        ],
        out_specs=pl.BlockSpec((ROW_BLK, N_CLASS), lambda i: (i, 0)),
        out_shape=jax.ShapeDtypeStruct((N, N_CLASS), jnp.float32),
        interpret=interpret,
    )(adj, yneigh, yself)
    return out


# speedup vs baseline: 1.4372x; 1.0257x over previous
import functools

import jax
import jax.numpy as jnp
from jax.experimental import pallas as pl
from jax.experimental.pallas import tpu as pltpu

N = 10000
D_IN = 128
D_HID = 128
N_CLASS = 16
ROW_BLK = 400
NB = N // ROW_BLK


def _fused_kernel(adj_ref, x_ref, xself_ref, w1_ref, b1_ref, w2_ref, b2_ref,
                  out_ref, yself_ref, yneigh_ref):
    i = pl.program_id(0)
    adj = adj_ref[...]
    deg = jnp.sum(adj, axis=1, keepdims=True)
    deg = jnp.maximum(deg, 1e-12)
    adj16 = adj.astype(jnp.bfloat16)
    blk = jax.lax.rem(i, NB)
    row = blk * ROW_BLK

    @pl.when(i < NB)
    def _layer1():
        acc = jax.lax.dot_general(
            adj16, x_ref[...].astype(jnp.bfloat16),
            (((1,), (0,)), ((), ())), preferred_element_type=jnp.float32)
        neigh = acc / deg
        w1 = w1_ref[...]
        h = (jax.lax.dot_general(xself_ref[...], w1[:D_IN],
                                 (((1,), (0,)), ((), ())),
                                 preferred_element_type=jnp.float32)
             + jax.lax.dot_general(neigh, w1[D_IN:],
                                   (((1,), (0,)), ((), ())),
                                   preferred_element_type=jnp.float32)
             + b1_ref[...])
        h = jnp.maximum(h, 0.0)
        w2 = w2_ref[...]
        yself_ref[pl.ds(row, ROW_BLK), :] = jax.lax.dot_general(
            h, w2[:D_HID], (((1,), (0,)), ((), ())),
            preferred_element_type=jnp.float32) + b2_ref[...]
        yneigh_ref[pl.ds(row, ROW_BLK), :] = jax.lax.dot_general(
            h, w2[D_HID:], (((1,), (0,)), ((), ())),
            preferred_element_type=jnp.float32)

    @pl.when(i >= NB)
    def _layer2():
        acc = jax.lax.dot_general(
            adj16, yneigh_ref[...].astype(jnp.bfloat16),
            (((1,), (0,)), ((), ())), preferred_element_type=jnp.float32)
        logits = yself_ref[pl.ds(row, ROW_BLK), :] + acc / deg
        m = jnp.max(logits, axis=1, keepdims=True)
        s = logits - m
        lse = jnp.log(jnp.sum(jnp.exp(s), axis=1, keepdims=True))
        out_ref[...] = s - lse


@functools.partial(jax.jit, static_argnames=("interpret",))
def kernel(feature, adj, W1, b1, W2, b2, interpret=False):
    b1r = b1.reshape(1, D_HID)
    b2r = b2.reshape(1, N_CLASS)

    out = pl.pallas_call(
        _fused_kernel,
        grid=(2 * NB,),
        in_specs=[
            pl.BlockSpec((ROW_BLK, N), lambda i: (jax.lax.rem(i, NB), 0)),
            pl.BlockSpec((N, D_IN), lambda i: (0, 0)),
            pl.BlockSpec((ROW_BLK, D_IN), lambda i: (jax.lax.rem(i, NB), 0)),
            pl.BlockSpec((2 * D_IN, D_HID), lambda i: (0, 0)),
            pl.BlockSpec((1, D_HID), lambda i: (0, 0)),
            pl.BlockSpec((2 * D_HID, N_CLASS), lambda i: (0, 0)),
            pl.BlockSpec((1, N_CLASS), lambda i: (0, 0)),
        ],
        out_specs=pl.BlockSpec((ROW_BLK, N_CLASS),
                               lambda i: (jax.lax.rem(i, NB), 0)),
        out_shape=jax.ShapeDtypeStruct((N, N_CLASS), jnp.float32),
        scratch_shapes=[
            pltpu.VMEM((N, N_CLASS), jnp.float32),
            pltpu.VMEM((N, N_CLASS), jnp.float32),
        ],
        interpret=interpret,
    )(adj, feature, feature, W1, b1r, W2, b2r)
    return out
